# baseline (speedup 1.0000x reference)
import jax
import jax.numpy as jnp
from jax import lax
from jax.experimental import pallas as pl
from jax.experimental.pallas import tpu as pltpu
from jax.experimental.pallas import tpu_sc as plsc

N = 10000
E = 320000
D = 128

NC = 2
NS = 16
NW = NC * NS
CH = 128
TOT = 80
NPAD = 10240
ROWS_PER_TILE = NPAD // NS
E_PAD = NW * CH * TOT
BLK = 256


def _pre_body(x_ref, w_ref, asrc_ref, adst_ref, h_ref, s_ref, d_ref,
              wself_ref, selfinit_ref):
    h = jnp.dot(x_ref[...], w_ref[...], preferred_element_type=jnp.float32)
    s = jnp.dot(h, asrc_ref[...], preferred_element_type=jnp.float32)
    d = jnp.dot(h, adst_ref[...], preferred_element_type=jnp.float32)
    e = s + d
    wself = jnp.exp(jnp.where(e >= 0, e, 0.2 * e))
    h_ref[...] = h
    s_ref[...] = s
    d_ref[...] = d
    wself_ref[...] = wself
    selfinit_ref[...] = wself * h


def _post_body(a0_ref, a1_ref, d0_ref, d1_ref, bias_ref, out_ref):
    den = d0_ref[...] + d1_ref[...] + 1e-16
    out_ref[...] = (a0_ref[...] + a1_ref[...]) / den + bias_ref[...]


def _sc_body(ei_flat, s_hbm, d_hbm, h_hbm, selfinit, zeros_nd, wselfp,
             zeros_n, acc_out, den_out, *refs):
    (w_a, w_b, rows_a, rows_b, sv_a, sv_b, dv_a, dv_b) = refs[:8]
    idx_refs = refs[8:8 + 4 * 2]
    acc_sh, den_sh, isem, gsem, ssem = refs[8 + 4 * 2:]
    idx = [[idx_refs[sl * 2 + t] for t in range(2)] for sl in range(4)]

    cid = lax.axis_index("c")
    sid = lax.axis_index("s")
    wid = sid * NC + cid
    rbase = sid * ROWS_PER_TILE
    cbase = wid * TOT

    @pl.when(cid == 0)
    def _():
        pltpu.sync_copy(selfinit.at[pl.ds(rbase, ROWS_PER_TILE)],
                        acc_sh.at[pl.ds(rbase, ROWS_PER_TILE)])
        pltpu.sync_copy(wselfp.at[pl.ds(rbase, ROWS_PER_TILE)],
                        den_sh.at[pl.ds(rbase, ROWS_PER_TILE)])

    @pl.when(cid != 0)
    def _():
        pltpu.sync_copy(zeros_nd.at[pl.ds(rbase, ROWS_PER_TILE)],
                        acc_sh.at[pl.ds(rbase, ROWS_PER_TILE)])
        pltpu.sync_copy(zeros_n.at[pl.ds(rbase, ROWS_PER_TILE)],
                        den_sh.at[pl.ds(rbase, ROWS_PER_TILE)])

    def eslice(c, t):
        return ei_flat.at[pl.ds((cbase + c) * (2 * CH) + t * CH, CH)]

    def idx_load_async(c, sl):
        for t in range(2):
            pltpu.async_copy(eslice(c, t), idx[sl][t], isem)

    def idx_load_wait(c, sl):
        for t in range(2):
            pltpu.make_async_copy(eslice(c, t), idx[sl][t], isem).wait()

    def idx_load_sync(c, sl):
        for t in range(2):
            pltpu.sync_copy(eslice(c, t), idx[sl][t])

    def gather(sl, rows, sv, dv):
        pltpu.async_copy(h_hbm.at[idx[sl][0]], rows, gsem)
        pltpu.async_copy(s_hbm.at[idx[sl][0]], sv, gsem)
        pltpu.async_copy(d_hbm.at[idx[sl][1]], dv, gsem)

    def drain_gather(sl, rows, sv, dv):
        pltpu.make_async_copy(h_hbm.at[idx[sl][0]], rows, gsem).wait()
        pltpu.make_async_copy(s_hbm.at[idx[sl][0]], sv, gsem).wait()
        pltpu.make_async_copy(d_hbm.at[idx[sl][1]], dv, gsem).wait()

    def scatter(sl, rows, w):
        pltpu.async_copy(rows, acc_sh.at[idx[sl][1]], ssem, add=True)
        pltpu.async_copy(w, den_sh.at[idx[sl][1]], ssem, add=True)

    def drain_scatter(sl, rows, w):
        pltpu.make_async_copy(rows, acc_sh.at[idx[sl][1]], ssem).wait()
        pltpu.make_async_copy(w, den_sh.at[idx[sl][1]], ssem).wait()

    bufs = [(rows_a, w_a, sv_a, dv_a), (rows_b, w_b, sv_b, dv_b)]

    idx_load_sync(0, 0)
    idx_load_sync(1, 1)
    gather(0, rows_a, sv_a, dv_a)

    def step(c, cc):
        sl = cc
        b = cc % 2
        rows, w, sv, dv = bufs[b]
        rows_p, w_p, sv_p, dv_p = bufs[1 - b]

        @pl.when(c >= 1)
        def _():
            drain_scatter((cc - 1) % 4, rows_p, w_p)
        drain_gather(sl, rows, sv, dv)

        @pl.when(c <= TOT - 3)
        def _():
            idx_load_async(c + 2, (cc + 2) % 4)

        @pl.when((c >= 1) & (c <= TOT - 2))
        def _():
            idx_load_wait(c + 1, (cc + 1) % 4)

        @pl.when(c <= TOT - 2)
        def _():
            gather((cc + 1) % 4, rows_p, sv_p, dv_p)

        for k in range(CH // 16):
            e = sv[pl.ds(k * 16, 16)] + dv[pl.ds(k * 16, 16)]
            e = jnp.where(e >= 0, e, 0.2 * e)
            w[pl.ds(k * 16, 16)] = jnp.exp(e)

        def mul_body(i, carry):
            wb = plsc.load_gather(w, [jnp.full((16,), i, jnp.int32)])
            for j in range(D // 16):
                rows[i, pl.ds(j * 16, 16)] = rows[i, pl.ds(j * 16, 16)] * wb
            return carry

        lax.fori_loop(0, CH, mul_body, 0, unroll=2)
        scatter(sl, rows, w)

    def loop_body(c4, carry):
        for cc in range(4):
            step(4 * c4 + cc, cc)
        return carry

    lax.fori_loop(0, TOT // 4, loop_body, 0)
    drain_scatter(3, rows_b, w_b)
    plsc.subcore_barrier()

    pltpu.sync_copy(acc_sh.at[pl.ds(rbase, ROWS_PER_TILE)],
                    acc_out.at[cid, pl.ds(rbase, ROWS_PER_TILE)])
    pltpu.sync_copy(den_sh.at[pl.ds(rbase, ROWS_PER_TILE)],
                    den_out.at[cid, pl.ds(rbase, ROWS_PER_TILE)])


@jax.jit
def kernel(x, edge_index, W, att_src, att_dst, bias):
    xp = jnp.zeros((NPAD, D), jnp.float32).at[:N].set(x)

    pre = pl.pallas_call(
        _pre_body,
        grid=(NPAD // BLK,),
        in_specs=[
            pl.BlockSpec((BLK, D), lambda i: (i, 0)),
            pl.BlockSpec((D, D), lambda i: (0, 0)),
            pl.BlockSpec((D, 1), lambda i: (0, 0)),
            pl.BlockSpec((D, 1), lambda i: (0, 0)),
        ],
        out_specs=[
            pl.BlockSpec((BLK, D), lambda i: (i, 0)),
            pl.BlockSpec((BLK, 1), lambda i: (i, 0)),
            pl.BlockSpec((BLK, 1), lambda i: (i, 0)),
            pl.BlockSpec((BLK, 1), lambda i: (i, 0)),
            pl.BlockSpec((BLK, D), lambda i: (i, 0)),
        ],
        out_shape=[
            jax.ShapeDtypeStruct((NPAD, D), jnp.float32),
            jax.ShapeDtypeStruct((NPAD, 1), jnp.float32),
            jax.ShapeDtypeStruct((NPAD, 1), jnp.float32),
            jax.ShapeDtypeStruct((NPAD, 1), jnp.float32),
            jax.ShapeDtypeStruct((NPAD, D), jnp.float32),
        ],
    )(xp, W, att_src.reshape(D, 1), att_dst.reshape(D, 1))
    h, s2, d2, wself2, selfinit = pre

    src_p = jnp.concatenate(
        [edge_index[0], jnp.zeros((E_PAD - E,), jnp.int32)])
    dst_p = jnp.concatenate(
        [edge_index[1], jnp.full((E_PAD - E,), N, jnp.int32)])
    ei_flat = jnp.stack([src_p.reshape(NW * TOT, CH),
                         dst_p.reshape(NW * TOT, CH)], axis=1).reshape(-1)

    idx_scratch = [pltpu.VMEM((CH,), jnp.int32) for _ in range(4 * 2)]
    sc_kernel = pl.kernel(
        _sc_body,
        out_type=[
            jax.ShapeDtypeStruct((NC, NPAD, D), jnp.float32),
            jax.ShapeDtypeStruct((NC, NPAD), jnp.float32),
        ],
        mesh=plsc.VectorSubcoreMesh(
            core_axis_name="c", subcore_axis_name="s",
            num_cores=NC, num_subcores=NS),
        compiler_params=pltpu.CompilerParams(needs_layout_passes=False),
        scratch_types=[
            pltpu.VMEM((CH,), jnp.float32),
            pltpu.VMEM((CH,), jnp.float32),
            pltpu.VMEM((CH, D), jnp.float32),
            pltpu.VMEM((CH, D), jnp.float32),
            pltpu.VMEM((CH,), jnp.float32),
            pltpu.VMEM((CH,), jnp.float32),
            pltpu.VMEM((CH,), jnp.float32),
            pltpu.VMEM((CH,), jnp.float32),
        ] + idx_scratch + [
            pltpu.VMEM_SHARED((NPAD, D), jnp.float32),
            pltpu.VMEM_SHARED((NPAD,), jnp.float32),
            pltpu.SemaphoreType.DMA,
            pltpu.SemaphoreType.DMA,
            pltpu.SemaphoreType.DMA,
        ],
    )
    acc, den = sc_kernel(
        ei_flat, s2.reshape(NPAD), d2.reshape(NPAD), h, selfinit,
        jnp.zeros((NPAD, D), jnp.float32), wself2.reshape(NPAD),
        jnp.zeros((NPAD,), jnp.float32))

    out = pl.pallas_call(
        _post_body,
        grid=(NPAD // BLK,),
        in_specs=[
            pl.BlockSpec((BLK, D), lambda i: (i, 0)),
            pl.BlockSpec((BLK, D), lambda i: (i, 0)),
            pl.BlockSpec((BLK, 1), lambda i: (i, 0)),
            pl.BlockSpec((BLK, 1), lambda i: (i, 0)),
            pl.BlockSpec((1, D), lambda i: (0, 0)),
        ],
        out_specs=pl.BlockSpec((BLK, D), lambda i: (i, 0)),
        out_shape=jax.ShapeDtypeStruct((NPAD, D), jnp.float32),
    )(acc[0], acc[1], den[0].reshape(NPAD, 1), den[1].reshape(NPAD, 1),
      bias.reshape(1, D))
    return out[:N]
